# baseline (device time: 60271 ns/iter reference)
import functools

import jax
import jax.numpy as jnp
from jax import lax
from jax.experimental import pallas as pl
from jax.experimental.pallas import tpu as pltpu

N_DEV = 32
N_STAGES = 5
EXPERTS_PER_DEV = 2


def kernel(x, router_W, route_idx, expert_W):
    del router_W
    n_tok, d_model = x.shape
    _, _, d_out = expert_W.shape

    def body(x_ref, idx_ref, w_ref, out_ref, accum_ref, comm_ref, send_sems, recv_sems):
        my = lax.axis_index("i")

        barrier = pltpu.get_barrier_semaphore()
        for k in range(N_STAGES):
            partner = my ^ (1 << k)
            pl.semaphore_signal(
                barrier, inc=1,
                device_id=(partner,), device_id_type=pl.DeviceIdType.MESH,
            )
        pl.semaphore_wait(barrier, N_STAGES)

        idx = idx_ref[...]
        xv = x_ref[...]
        e0 = my * EXPERTS_PER_DEV
        acc = jnp.zeros((n_tok, d_out), dtype=jnp.float32)
        for j in range(EXPERTS_PER_DEV):
            xm = jnp.where(idx == e0 + j, xv, 0.0).astype(jnp.bfloat16)
            acc = acc + jnp.dot(
                xm, w_ref[j].astype(jnp.bfloat16),
                preferred_element_type=jnp.float32,
            )
        accum_ref[...] = acc.astype(jnp.bfloat16)

        for k in range(N_STAGES):
            partner = my ^ (1 << k)
            rdma = pltpu.make_async_remote_copy(
                src_ref=accum_ref,
                dst_ref=comm_ref.at[k],
                send_sem=send_sems.at[k],
                recv_sem=recv_sems.at[k],
                device_id=(partner,),
                device_id_type=pl.DeviceIdType.MESH,
            )
            rdma.start()
            rdma.wait()
            accum_ref[...] = accum_ref[...] + comm_ref[k]

        out_ref[...] = accum_ref[...].astype(jnp.float32)

        @functools.partial(pl.run_scoped, sem2=pltpu.SemaphoreType.REGULAR)
        def _(sem2):
            for k in range(N_STAGES):
                partner = my ^ (1 << k)
                pl.semaphore_signal(
                    sem2, inc=1,
                    device_id=(partner,), device_id_type=pl.DeviceIdType.MESH,
                )
            pl.semaphore_wait(sem2, N_STAGES)

    return pl.pallas_call(
        body,
        out_shape=jax.ShapeDtypeStruct((n_tok, d_out), jnp.float32),
        in_specs=[
            pl.BlockSpec(memory_space=pltpu.VMEM),
            pl.BlockSpec(memory_space=pltpu.VMEM),
            pl.BlockSpec(memory_space=pltpu.VMEM),
        ],
        out_specs=pl.BlockSpec(memory_space=pltpu.VMEM),
        scratch_shapes=[
            pltpu.VMEM((n_tok, d_out), jnp.bfloat16),
            pltpu.VMEM((N_STAGES, n_tok, d_out), jnp.bfloat16),
            pltpu.SemaphoreType.DMA((N_STAGES,)),
            pltpu.SemaphoreType.DMA((N_STAGES,)),
        ],
        compiler_params=pltpu.CompilerParams(collective_id=0),
    )(x, route_idx, expert_W)


# device time: 29174 ns/iter; 2.0659x vs baseline; 2.0659x over previous
import jax
import jax.numpy as jnp
from jax import lax
from jax.experimental import pallas as pl
from jax.experimental.pallas import tpu as pltpu

N_DEV = 32
EXPERTS_PER_DEV = 2


def kernel(x, router_W, route_idx, expert_W):
    del router_W
    n_tok, d_model = x.shape
    _, _, d_out = expert_W.shape
    blk = n_tok // N_DEV

    def body(x_ref, idx_ref, w_ref, out_ref,
             partial_ref, slab_ref, accum_ref,
             rs_send_sems, rs_recv_sems, ag_send_sems, ag_recv_sems):
        my = lax.axis_index("i")

        barrier = pltpu.get_barrier_semaphore()
        for o in range(1, N_DEV):
            pl.semaphore_signal(
                barrier, inc=1,
                device_id=((my + o) % N_DEV,),
                device_id_type=pl.DeviceIdType.MESH,
            )
        pl.semaphore_wait(barrier, N_DEV - 1)

        idx = idx_ref[...]
        xv = x_ref[...]
        e0 = my * EXPERTS_PER_DEV
        acc = jnp.zeros((n_tok, d_out), dtype=jnp.float32)
        for j in range(EXPERTS_PER_DEV):
            xm = jnp.where(idx == e0 + j, xv, 0.0).astype(jnp.bfloat16)
            acc = acc + jnp.dot(
                xm, w_ref[j].astype(jnp.bfloat16),
                preferred_element_type=jnp.float32,
            )
        partial_ref[...] = acc.astype(jnp.bfloat16)

        rs_sends = []
        for o in range(1, N_DEV):
            dst = (my + o) % N_DEV
            rdma = pltpu.make_async_remote_copy(
                src_ref=partial_ref.at[pl.ds(dst * blk, blk)],
                dst_ref=slab_ref.at[o],
                send_sem=rs_send_sems.at[o],
                recv_sem=rs_recv_sems.at[o],
                device_id=(dst,),
                device_id_type=pl.DeviceIdType.MESH,
            )
            rdma.start()
            rs_sends.append(rdma)
        slab_ref[0] = partial_ref[pl.ds(my * blk, blk), :]

        for rdma in rs_sends:
            rdma.wait_recv()

        block_sum = jnp.sum(slab_ref[...].astype(jnp.float32), axis=0)
        accum_ref[pl.ds(my * blk, blk), :] = block_sum.astype(jnp.bfloat16)

        ag_sends = []
        for o in range(1, N_DEV):
            dst = (my + o) % N_DEV
            rdma = pltpu.make_async_remote_copy(
                src_ref=accum_ref.at[pl.ds(my * blk, blk)],
                dst_ref=accum_ref.at[pl.ds(my * blk, blk)],
                send_sem=ag_send_sems.at[o],
                recv_sem=ag_recv_sems.at[o],
                device_id=(dst,),
                device_id_type=pl.DeviceIdType.MESH,
            )
            rdma.start()
            ag_sends.append(rdma)

        for rdma in rs_sends:
            rdma.wait_send()
        for rdma in ag_sends:
            rdma.wait_recv()
        for rdma in ag_sends:
            rdma.wait_send()

        out_ref[...] = accum_ref[...].astype(jnp.float32)

    return pl.pallas_call(
        body,
        out_shape=jax.ShapeDtypeStruct((n_tok, d_out), jnp.float32),
        in_specs=[
            pl.BlockSpec(memory_space=pltpu.VMEM),
            pl.BlockSpec(memory_space=pltpu.VMEM),
            pl.BlockSpec(memory_space=pltpu.VMEM),
        ],
        out_specs=pl.BlockSpec(memory_space=pltpu.VMEM),
        scratch_shapes=[
            pltpu.VMEM((n_tok, d_out), jnp.bfloat16),
            pltpu.VMEM((N_DEV, blk, d_out), jnp.bfloat16),
            pltpu.VMEM((n_tok, d_out), jnp.bfloat16),
            pltpu.SemaphoreType.DMA((N_DEV,)),
            pltpu.SemaphoreType.DMA((N_DEV,)),
            pltpu.SemaphoreType.DMA((N_DEV,)),
            pltpu.SemaphoreType.DMA((N_DEV,)),
        ],
        compiler_params=pltpu.CompilerParams(collective_id=0),
    )(x, route_idx, expert_W)
